# baseline (speedup 1.0000x reference)
import jax
import jax.numpy as jnp
from jax import lax
from jax.experimental import pallas as pl
from jax.experimental.pallas import tpu as pltpu
from jax.experimental.pallas import tpu_sc as plsc

D = 64
DP = 128
NC, NS = 2, 16
NW = NC * NS
CH = 128
NBUF = 4


def _make_kernel(B):
    assert B % (NW * CH) == 0
    S = B // (NW * CH)
    mesh = plsc.VectorSubcoreMesh(core_axis_name="c", subcore_axis_name="s")

    def body(x_hbm, table_hbm, out_hbm, idx_f, idx_v, rows_v, gsem, ssem):
        wid = lax.axis_index("s") * NC + lax.axis_index("c")
        base = wid * (S * CH)
        pltpu.sync_copy(x_hbm.at[pl.ds(wid * S, S)], idx_f)

        def conv(s, carry):
            for k in range(CH // 16):
                idx_v[s, pl.ds(k * 16, 16)] = plsc.bitcast(
                    idx_f[s, pl.ds(k * 16, 16)], jnp.int32)
            return carry

        lax.fori_loop(0, S, conv, 0)

        def g_copy(t):
            b = lax.rem(t, NBUF)
            return pltpu.make_async_copy(
                table_hbm.at[idx_v.at[t]],
                rows_v.at[b], gsem.at[b])

        def s_copy(t):
            b = lax.rem(t, NBUF)
            return pltpu.make_async_copy(
                rows_v.at[b], out_hbm.at[pl.ds(base + t * CH, CH)],
                ssem.at[b])

        def step(t, carry):
            @pl.when(t >= NBUF)
            def _():
                s_copy(t - NBUF).wait()

            @pl.when(t < S)
            def _():
                g_copy(t).start()

            u = t - (NBUF - 1)

            @pl.when(jnp.logical_and(u >= 0, u < S))
            def _():
                g_copy(u).wait()
                s_copy(u).start()

            return carry

        lax.fori_loop(0, S + NBUF - 1, step, 0)
        s_copy(S - 1).wait()

    return pl.kernel(
        body,
        out_type=jax.ShapeDtypeStruct((B, DP), jnp.float32),
        mesh=mesh,
        compiler_params=pltpu.CompilerParams(
            use_tc_tiling_on_sc=False, needs_layout_passes=False),
        scratch_types=[
            pltpu.VMEM((S, CH), jnp.float32),
            pltpu.VMEM((S, CH), jnp.int32),
            pltpu.VMEM((NBUF, CH, DP), jnp.float32),
            pltpu.SemaphoreType.DMA((NBUF,)),
            pltpu.SemaphoreType.DMA((NBUF,)),
        ],
    )


def kernel(x, table):
    NR, J = x.shape
    B = NR * J
    x2 = lax.bitcast_convert_type(
        x.reshape(B // CH, CH).astype(jnp.int32), jnp.float32)
    tp = jnp.pad(table, ((0, 0), (0, DP - D)))
    out = _make_kernel(B)(x2, tp)
    return out[:, :D].reshape(NR, J, D)


# speedup vs baseline: 1.0857x; 1.0857x over previous
import jax
import jax.numpy as jnp
from jax import lax
from jax.experimental import pallas as pl
from jax.experimental.pallas import tpu as pltpu
from jax.experimental.pallas import tpu_sc as plsc

D = 64
DP = 128
NC, NS = 2, 16
NW = NC * NS
J = 200
JP = 256
J1 = 128
J2 = J - J1
NBUF = 3


def _make_kernel(NR):
    assert NR % NW == 0
    R = NR // NW
    mesh = plsc.VectorSubcoreMesh(core_axis_name="c", subcore_axis_name="s")

    def body(x_hbm, table_hbm, out_hbm, idx_v, rows_v, gsem, ssem):
        wid = lax.axis_index("s") * NC + lax.axis_index("c")
        base = wid * R
        pltpu.sync_copy(x_hbm.at[pl.ds(base * 2, R * 2)], idx_v)

        def g_start(t):
            b = lax.rem(t, NBUF)
            pltpu.make_async_copy(
                table_hbm.at[idx_v.at[2 * t]],
                rows_v.at[b, pl.ds(0, J1)], gsem.at[b]).start()
            pltpu.make_async_copy(
                table_hbm.at[idx_v.at[2 * t + 1, pl.ds(0, J2)]],
                rows_v.at[b, pl.ds(J1, J2)], gsem.at[b]).start()

        def g_wait(t):
            b = lax.rem(t, NBUF)
            pltpu.make_async_copy(
                table_hbm.at[pl.ds(0, J)], rows_v.at[b], gsem.at[b]).wait()

        def s_copy(t):
            b = lax.rem(t, NBUF)
            return pltpu.make_async_copy(
                rows_v.at[b, :, pl.ds(0, D)],
                out_hbm.at[pl.ds((base + t) * J, J), pl.ds(0, D)],
                ssem.at[b])

        def step(t, carry):
            @pl.when(t >= NBUF)
            def _():
                s_copy(t - NBUF).wait()

            @pl.when(t < R)
            def _():
                g_start(t)

            u = t - (NBUF - 1)

            @pl.when(jnp.logical_and(u >= 0, u < R))
            def _():
                g_wait(u)
                s_copy(u).start()

            return carry

        lax.fori_loop(0, R + NBUF - 1, step, 0)
        s_copy(R - 1).wait()

    return pl.kernel(
        body,
        out_type=jax.ShapeDtypeStruct((NR * J, DP), jnp.float32),
        mesh=mesh,
        compiler_params=pltpu.CompilerParams(use_tc_tiling_on_sc=False),
        scratch_types=[
            pltpu.VMEM((2 * R, J1), jnp.int32),
            pltpu.VMEM((NBUF, J, DP), jnp.float32),
            pltpu.SemaphoreType.DMA((NBUF,)),
            pltpu.SemaphoreType.DMA((NBUF,)),
        ],
    )


def kernel(x, table):
    NR = x.shape[0]
    xp = jnp.pad(x.astype(jnp.int32), ((0, 0), (0, JP - J)))
    x2 = xp.reshape(NR * 2, J1)
    tp = jnp.pad(table, ((0, 0), (0, DP - D)))
    out = _make_kernel(NR)(x2, tp)
    return out[:, :D].reshape(NR, J, D)


# speedup vs baseline: 1.0944x; 1.0080x over previous
import jax
import jax.numpy as jnp
from jax import lax
from jax.experimental import pallas as pl
from jax.experimental.pallas import tpu as pltpu
from jax.experimental.pallas import tpu_sc as plsc

D = 64
DP = 128
NC, NS = 2, 16
NW = NC * NS
J = 200
JP = 256
J1 = 128
J2 = J - J1
NBUF = 4


def _make_kernel(NR):
    assert NR % NW == 0
    R = NR // NW
    mesh = plsc.VectorSubcoreMesh(core_axis_name="c", subcore_axis_name="s")

    def body(x_hbm, table_hbm, out_hbm, idx_v, rows_v, gsem, ssem):
        wid = lax.axis_index("s") * NC + lax.axis_index("c")
        base = wid * R
        pltpu.sync_copy(x_hbm.at[pl.ds(base * 2, R * 2)], idx_v)

        def g_start(t):
            b = lax.rem(t, NBUF)
            pltpu.make_async_copy(
                table_hbm.at[idx_v.at[2 * t]],
                rows_v.at[b, pl.ds(0, J1)], gsem.at[b]).start()
            pltpu.make_async_copy(
                table_hbm.at[idx_v.at[2 * t + 1, pl.ds(0, J2)]],
                rows_v.at[b, pl.ds(J1, J2)], gsem.at[b]).start()

        def g_wait(t):
            b = lax.rem(t, NBUF)
            pltpu.make_async_copy(
                table_hbm.at[pl.ds(0, J)], rows_v.at[b], gsem.at[b]).wait()

        def s_copy(t):
            b = lax.rem(t, NBUF)
            return pltpu.make_async_copy(
                rows_v.at[b],
                out_hbm.at[pl.ds((base + t) * J, J), pl.ds(0, D)],
                ssem.at[b])

        def step(t, carry):
            @pl.when(t >= NBUF)
            def _():
                s_copy(t - NBUF).wait()

            @pl.when(t < R)
            def _():
                g_start(t)

            u = t - (NBUF - 1)

            @pl.when(jnp.logical_and(u >= 0, u < R))
            def _():
                g_wait(u)
                s_copy(u).start()

            return carry

        lax.fori_loop(0, R + NBUF - 1, step, 0)
        s_copy(R - 1).wait()

    return pl.kernel(
        body,
        out_type=jax.ShapeDtypeStruct((NR * J, DP), jnp.float32),
        mesh=mesh,
        compiler_params=pltpu.CompilerParams(use_tc_tiling_on_sc=False),
        scratch_types=[
            pltpu.VMEM((2 * R, J1), jnp.int32),
            pltpu.VMEM((NBUF, J, D), jnp.float32),
            pltpu.SemaphoreType.DMA((NBUF,)),
            pltpu.SemaphoreType.DMA((NBUF,)),
        ],
    )


def kernel(x, table):
    NR = x.shape[0]
    xp = jnp.pad(x.astype(jnp.int32), ((0, 0), (0, JP - J)))
    x2 = xp.reshape(NR * 2, J1)
    out = _make_kernel(NR)(x2, table)
    return out[:, :D].reshape(NR, J, D)
